# baseline (speedup 1.0000x reference)
import jax
import jax.numpy as jnp
from jax import lax
from jax.experimental import pallas as pl
from jax.experimental.pallas import tpu as pltpu

E = 8
H = 1024
I = 2048
BT = 256
TPE = 8
NT = 24
ROWS = (NT + 1) * BT
TBLK = 256
MM_DTYPE = jnp.bfloat16


def _router_body(x_ref, gw_ref, sel_ref, wv_ref):
    xb = x_ref[...]
    gw = gw_ref[...]
    l = lax.dot_general(xb, gw, (((1,), (1,)), ((), ())),
                        preferred_element_type=jnp.float32)
    m = jnp.max(l, axis=1, keepdims=True)
    q = jnp.exp(l - m)
    ii = lax.broadcasted_iota(jnp.int32, l.shape, 1)
    m1 = jnp.max(q, axis=1, keepdims=True)
    i1 = jnp.min(jnp.where(q == m1, ii, E), axis=1, keepdims=True)
    q2 = jnp.where(ii == i1, -1.0, q)
    m2 = jnp.max(q2, axis=1, keepdims=True)
    i2 = jnp.min(jnp.where(q2 == m2, ii, E), axis=1, keepdims=True)
    s = m1 + m2
    sel_ref[...] = jnp.where((ii == i1) | (ii == i2), 1.0, 0.0)
    wv_ref[...] = (jnp.where(ii == i1, m1 / s, 0.0)
                   + jnp.where(ii == i2, m2 / s, 0.0))


def _ffn_body(offt_ref, tl_ref, xs_ref, w1_ref, w3_ref, w2_ref, yin_ref,
              ys_ref):
    del offt_ref, yin_ref
    j = pl.program_id(0)

    @pl.when(j < tl_ref[0])
    def _():
        xb = xs_ref[...]
        w1 = w1_ref[0]
        w3 = w3_ref[0]
        h1 = lax.dot_general(xb, w1, (((1,), (1,)), ((), ())),
                             preferred_element_type=jnp.float32)
        h3 = lax.dot_general(xb, w3, (((1,), (1,)), ((), ())),
                             preferred_element_type=jnp.float32)
        act = (h1 * lax.logistic(h1) * h3).astype(MM_DTYPE)
        w2 = w2_ref[0]
        ys_ref[...] = lax.dot_general(act, w2, (((1,), (1,)), ((), ())),
                                      preferred_element_type=jnp.float32)


def kernel(index, hidden_states, experts_cache, gate_w, ws):
    x = hidden_states
    T = x.shape[0]
    del index
    gw = gate_w.reshape(E, H)

    sel, wv = pl.pallas_call(
        _router_body,
        grid=(T // TBLK,),
        in_specs=[pl.BlockSpec((TBLK, H), lambda i: (i, 0)),
                  pl.BlockSpec((E, H), lambda i: (0, 0))],
        out_specs=[pl.BlockSpec((TBLK, E), lambda i: (i, 0)),
                   pl.BlockSpec((TBLK, E), lambda i: (i, 0))],
        out_shape=[jax.ShapeDtypeStruct((T, E), jnp.float32),
                   jax.ShapeDtypeStruct((T, E), jnp.float32)],
    )(x, gw)

    selb = sel > 0.5
    seli = selb.astype(jnp.int32)
    cnt = jnp.sum(seli, axis=0)
    pos = jnp.cumsum(seli, axis=0) - seli
    tiles = ((cnt + BT - 1) // BT).astype(jnp.int32)
    offt = (jnp.cumsum(tiles) - tiles).astype(jnp.int32)
    dest = offt[None, :] * BT + pos
    tok = lax.broadcasted_iota(jnp.int32, (T, E), 0)
    flat_dest = jnp.where(selb, dest, ROWS)
    src_rows = (jnp.zeros((ROWS + 1,), jnp.int32)
                .at[flat_dest.reshape(-1)].set(tok.reshape(-1))[:ROWS])
    ord_ = jnp.argsort(1 - seli, axis=1, stable=True)
    e0 = ord_[:, 0]
    e1 = ord_[:, 1]
    ar = jnp.arange(T)
    r0 = dest[ar, e0]
    r1 = dest[ar, e1]
    w0 = wv[ar, e0]
    w1_ = wv[ar, e1]

    xs = jnp.take(x.astype(MM_DTYPE), src_rows, axis=0)
    wsb = ws.astype(MM_DTYPE)
    wsa = wsb.reshape(E, 3 * I, H)
    wsw2 = wsb.reshape(E, 3 * H, I)
    cache = experts_cache.astype(jnp.int32)
    wa_e = [lax.dynamic_slice(wsa, (cache[e], 0, 0), (1, 2 * I, H))
            for e in range(E)]
    wb_e = [lax.dynamic_slice(wsw2, (cache[e], 2 * H, 0), (1, H, I))
            for e in range(E)]

    ys = jnp.zeros((ROWS, H), jnp.float32)

    def xs_idx(j, offt_r, tl_r):
        return (jnp.where(j < tl_r[0], offt_r[0] + j, NT), 0)

    def w_idx(j, offt_r, tl_r):
        return (0, 0, 0)

    def w3_idx(j, offt_r, tl_r):
        return (0, 1, 0)

    for e in range(E):
        offt_e = offt[e][None]
        tl_e = tiles[e][None]
        grid_spec = pltpu.PrefetchScalarGridSpec(
            num_scalar_prefetch=2,
            grid=(TPE,),
            in_specs=[
                pl.BlockSpec((BT, H), xs_idx),
                pl.BlockSpec((1, I, H), w_idx),
                pl.BlockSpec((1, I, H), w3_idx),
                pl.BlockSpec((1, H, I), w_idx),
                pl.BlockSpec((BT, H), xs_idx),
            ],
            out_specs=pl.BlockSpec((BT, H), xs_idx),
        )
        ys = pl.pallas_call(
            _ffn_body,
            grid_spec=grid_spec,
            out_shape=jax.ShapeDtypeStruct((ROWS, H), jnp.float32),
            input_output_aliases={6: 0},
            compiler_params=pltpu.CompilerParams(
                dimension_semantics=("arbitrary",)),
        )(offt_e, tl_e, xs, wa_e[e], wa_e[e], wb_e[e], ys)

    out = w0[:, None] * ys[r0] + w1_[:, None] * ys[r1]
    return out


# speedup vs baseline: 1.1347x; 1.1347x over previous
import functools

import jax
import jax.numpy as jnp
from jax import lax
from jax.experimental import pallas as pl
from jax.experimental.pallas import tpu as pltpu

E = 8
H = 1024
I = 2048
BT = 256
NT = 24
ROWS = NT * BT
TBLK = 256
MM_DTYPE = jnp.bfloat16


def _router_body(x_ref, gw_ref, sel_ref, wv_ref):
    xb = x_ref[...]
    gw = gw_ref[...]
    l = lax.dot_general(xb, gw, (((1,), (1,)), ((), ())),
                        preferred_element_type=jnp.float32)
    m = jnp.max(l, axis=1, keepdims=True)
    q = jnp.exp(l - m)
    ii = lax.broadcasted_iota(jnp.int32, l.shape, 1)
    m1 = jnp.max(q, axis=1, keepdims=True)
    i1 = jnp.min(jnp.where(q == m1, ii, E), axis=1, keepdims=True)
    q2 = jnp.where(ii == i1, -1.0, q)
    m2 = jnp.max(q2, axis=1, keepdims=True)
    i2 = jnp.min(jnp.where(q2 == m2, ii, E), axis=1, keepdims=True)
    s = m1 + m2
    sel_ref[...] = jnp.where((ii == i1) | (ii == i2), 1.0, 0.0)
    wv_ref[...] = (jnp.where(ii == i1, m1 / s, 0.0)
                   + jnp.where(ii == i2, m2 / s, 0.0))


def _ffn_body(te_ref, nv_ref, ca_ref, xs_ref, w1_ref, w3_ref, w2_ref, ys_ref):
    i = pl.program_id(0)

    @pl.when(i < nv_ref[0])
    def _():
        xb = xs_ref[...]
        w1 = w1_ref[0]
        w3 = w3_ref[0]
        h1 = lax.dot_general(xb, w1, (((1,), (1,)), ((), ())),
                             preferred_element_type=jnp.float32)
        h3 = lax.dot_general(xb, w3, (((1,), (1,)), ((), ())),
                             preferred_element_type=jnp.float32)
        act = (h1 * lax.logistic(h1) * h3).astype(MM_DTYPE)
        w2 = w2_ref[0]
        ys_ref[...] = lax.dot_general(act, w2, (((1,), (1,)), ((), ())),
                                      preferred_element_type=jnp.float32)


def kernel(index, hidden_states, experts_cache, gate_w, ws):
    x = hidden_states
    T = x.shape[0]
    del index
    gw = gate_w.reshape(E, H)

    sel, wv = pl.pallas_call(
        _router_body,
        grid=(T // TBLK,),
        in_specs=[pl.BlockSpec((TBLK, H), lambda i: (i, 0)),
                  pl.BlockSpec((E, H), lambda i: (0, 0))],
        out_specs=[pl.BlockSpec((TBLK, E), lambda i: (i, 0)),
                   pl.BlockSpec((TBLK, E), lambda i: (i, 0))],
        out_shape=[jax.ShapeDtypeStruct((T, E), jnp.float32),
                   jax.ShapeDtypeStruct((T, E), jnp.float32)],
    )(x, gw)

    selb = sel > 0.5
    seli = selb.astype(jnp.int32)
    cnt = jnp.sum(seli, axis=0)
    pos = jnp.cumsum(seli, axis=0) - seli
    tiles = (cnt + BT - 1) // BT
    offt = (jnp.cumsum(tiles) - tiles).astype(jnp.int32)
    nvalid = jnp.sum(tiles).astype(jnp.int32)
    dest = offt[None, :] * BT + pos
    tok = lax.broadcasted_iota(jnp.int32, (T, E), 0)
    flat_dest = jnp.where(selb, dest, ROWS)
    src_rows = (jnp.zeros((ROWS + 1,), jnp.int32)
                .at[flat_dest.reshape(-1)].set(tok.reshape(-1))[:ROWS])
    ord_ = jnp.argsort(1 - seli, axis=1, stable=True)
    e0 = ord_[:, 0]
    e1 = ord_[:, 1]
    ar = jnp.arange(T)
    r0 = dest[ar, e0]
    r1 = dest[ar, e1]
    w0 = wv[ar, e0]
    w1_ = wv[ar, e1]
    ti = jnp.arange(NT, dtype=jnp.int32)
    tile_e = jnp.clip(jnp.searchsorted(offt, ti, side='right') - 1,
                      0, E - 1).astype(jnp.int32)

    xs = jnp.take(x.astype(MM_DTYPE), src_rows, axis=0)
    wsf = ws.reshape(1, E, 3 * I * H)
    w1v = wsf[:, :, :I * H].astype(MM_DTYPE).reshape(E, I, H)
    w3v = wsf[:, :, I * H:2 * I * H].astype(MM_DTYPE).reshape(E, I, H)
    w2v = wsf[:, :, 2 * I * H:].astype(MM_DTYPE).reshape(E, H, I)
    cache = experts_cache.astype(jnp.int32)

    grid_spec = pltpu.PrefetchScalarGridSpec(
        num_scalar_prefetch=3,
        grid=(NT,),
        in_specs=[
            pl.BlockSpec((BT, H), lambda i, te, nv, ca: (i, 0)),
            pl.BlockSpec((1, I, H), lambda i, te, nv, ca: (ca[te[i]], 0, 0)),
            pl.BlockSpec((1, I, H), lambda i, te, nv, ca: (ca[te[i]], 0, 0)),
            pl.BlockSpec((1, H, I), lambda i, te, nv, ca: (ca[te[i]], 0, 0)),
        ],
        out_specs=pl.BlockSpec((BT, H), lambda i, te, nv, ca: (i, 0)),
    )
    ys = pl.pallas_call(
        _ffn_body,
        grid_spec=grid_spec,
        out_shape=jax.ShapeDtypeStruct((ROWS, H), jnp.float32),
        compiler_params=pltpu.CompilerParams(
            dimension_semantics=("arbitrary",)),
    )(tile_e, jnp.reshape(nvalid, (1,)), cache, xs, w1v, w3v, w2v)

    out = w0[:, None] * ys[r0] + w1_[:, None] * ys[r1]
    return out


# speedup vs baseline: 1.1823x; 1.0420x over previous
import functools

import jax
import jax.numpy as jnp
from jax import lax
from jax.experimental import pallas as pl
from jax.experimental.pallas import tpu as pltpu

E = 8
H = 1024
I = 2048
BT = 256
NT = 24
ROWS = NT * BT
TBLK = 256
MM_DTYPE = jnp.bfloat16


def _router_body(x_ref, gw_ref, sel_ref, i1_ref, i2_ref, w0_ref, w1_ref):
    xb = x_ref[...]
    gw = gw_ref[...]
    l = lax.dot_general(xb, gw, (((1,), (1,)), ((), ())),
                        preferred_element_type=jnp.float32)
    m = jnp.max(l, axis=1, keepdims=True)
    q = jnp.exp(l - m)
    ii = lax.broadcasted_iota(jnp.int32, l.shape, 1)
    m1 = jnp.max(q, axis=1, keepdims=True)
    i1 = jnp.min(jnp.where(q == m1, ii, E), axis=1, keepdims=True)
    q2 = jnp.where(ii == i1, -1.0, q)
    m2 = jnp.max(q2, axis=1, keepdims=True)
    i2 = jnp.min(jnp.where(q2 == m2, ii, E), axis=1, keepdims=True)
    s = m1 + m2
    sel_ref[...] = jnp.where((ii == i1) | (ii == i2), 1.0, 0.0)
    i1_ref[...] = jnp.broadcast_to(i1, l.shape)
    i2_ref[...] = jnp.broadcast_to(i2, l.shape)
    w0_ref[...] = jnp.broadcast_to(m1 / s, l.shape)
    w1_ref[...] = jnp.broadcast_to(m2 / s, l.shape)


def _ffn_body(te_ref, nv_ref, ca_ref, xs_ref, w1_ref, w3_ref, w2_ref, ys_ref):
    i = pl.program_id(0)

    @pl.when(i < nv_ref[0])
    def _():
        xb = xs_ref[...]
        w1 = w1_ref[0]
        w3 = w3_ref[0]
        h1 = lax.dot_general(xb, w1, (((1,), (1,)), ((), ())),
                             preferred_element_type=jnp.float32)
        h3 = lax.dot_general(xb, w3, (((1,), (1,)), ((), ())),
                             preferred_element_type=jnp.float32)
        act = (h1 * lax.logistic(h1) * h3).astype(MM_DTYPE)
        w2 = w2_ref[0]
        ys_ref[...] = lax.dot_general(act, w2, (((1,), (1,)), ((), ())),
                                      preferred_element_type=jnp.float32)


def kernel(index, hidden_states, experts_cache, gate_w, ws):
    x = hidden_states
    T = x.shape[0]
    del index
    gw = gate_w.reshape(E, H)

    blk = pl.BlockSpec((TBLK, E), lambda i: (i, 0))
    sel, i1o, i2o, w0o, w1o = pl.pallas_call(
        _router_body,
        grid=(T // TBLK,),
        in_specs=[pl.BlockSpec((TBLK, H), lambda i: (i, 0)),
                  pl.BlockSpec((E, H), lambda i: (0, 0))],
        out_specs=[blk, blk, blk, blk, blk],
        out_shape=[jax.ShapeDtypeStruct((T, E), jnp.float32),
                   jax.ShapeDtypeStruct((T, E), jnp.int32),
                   jax.ShapeDtypeStruct((T, E), jnp.int32),
                   jax.ShapeDtypeStruct((T, E), jnp.float32),
                   jax.ShapeDtypeStruct((T, E), jnp.float32)],
    )(x, gw)

    selb = sel > 0.5
    seli = selb.astype(jnp.int32)
    cnt = jnp.sum(seli, axis=0)
    pos = jnp.cumsum(seli, axis=0) - seli
    tiles = (cnt + BT - 1) // BT
    offt = (jnp.cumsum(tiles) - tiles).astype(jnp.int32)
    nvalid = jnp.sum(tiles).astype(jnp.int32)
    ar = jnp.arange(T)
    e0 = i1o[:, 0]
    e1 = i2o[:, 0]
    w0 = w0o[:, 0]
    w1_ = w1o[:, 0]
    r0 = offt[e0] * BT + pos[ar, e0]
    r1 = offt[e1] * BT + pos[ar, e1]
    src_rows = (jnp.zeros((ROWS,), jnp.int32)
                .at[jnp.concatenate([r0, r1])]
                .set(jnp.concatenate([ar, ar]).astype(jnp.int32)))
    ti = jnp.arange(NT, dtype=jnp.int32)
    tile_e = jnp.clip(jnp.searchsorted(offt, ti, side='right') - 1,
                      0, E - 1).astype(jnp.int32)

    xs = jnp.take(x.astype(MM_DTYPE), src_rows, axis=0)
    wsf = ws.reshape(1, E, 3 * I * H)
    w1v = wsf[:, :, :I * H].astype(MM_DTYPE).reshape(E, I, H)
    w3v = wsf[:, :, I * H:2 * I * H].astype(MM_DTYPE).reshape(E, I, H)
    w2v = wsf[:, :, 2 * I * H:].astype(MM_DTYPE).reshape(E, H, I)
    cache = experts_cache.astype(jnp.int32)

    grid_spec = pltpu.PrefetchScalarGridSpec(
        num_scalar_prefetch=3,
        grid=(NT,),
        in_specs=[
            pl.BlockSpec((BT, H), lambda i, te, nv, ca: (i, 0)),
            pl.BlockSpec((1, I, H), lambda i, te, nv, ca: (ca[te[i]], 0, 0)),
            pl.BlockSpec((1, I, H), lambda i, te, nv, ca: (ca[te[i]], 0, 0)),
            pl.BlockSpec((1, H, I), lambda i, te, nv, ca: (ca[te[i]], 0, 0)),
        ],
        out_specs=pl.BlockSpec((BT, H), lambda i, te, nv, ca: (i, 0)),
    )
    ys = pl.pallas_call(
        _ffn_body,
        grid_spec=grid_spec,
        out_shape=jax.ShapeDtypeStruct((ROWS, H), jnp.float32),
        compiler_params=pltpu.CompilerParams(
            dimension_semantics=("arbitrary",)),
    )(tile_e, jnp.reshape(nvalid, (1,)), cache, xs, w1v, w3v, w2v)

    out = w0[:, None] * ys[r0] + w1_[:, None] * ys[r1]
    return out
